# Initial kernel scaffold; baseline (speedup 1.0000x reference)
import jax
import jax.numpy as jnp
from jax.experimental import pallas as pl


def kernel(x, edge_index, W1, b1, W2, b2):
    raise NotImplementedError("write your pallas kernel here")


# speedup vs baseline: 93.1779x; 93.1779x over previous
import functools

import jax
import jax.numpy as jnp
from jax import lax
from jax.experimental import pallas as pl
from jax.experimental.pallas import tpu as pltpu
from jax.experimental.pallas import tpu_sc as plsc

N = 100000
E = 6400000
NP = 100096
B = 128
ROWS = E // B
NC = 2
NS = 16
NW = NC * NS
RB = 8
NBT = ROWS // RB
NBF = NBT // NW
NBR = NBT - NBF * NW
STRIPE = NP // NS

_mesh = plsc.VectorSubcoreMesh(core_axis_name="c", subcore_axis_name="s")


def _make_sc_agg(ncol):
    @functools.partial(
        pl.kernel,
        out_type=[jax.ShapeDtypeStruct((NC * NP,), jnp.float32)] * ncol,
        mesh=_mesh,
        scratch_types=[
            pltpu.VMEM((RB, B), jnp.int32),
            pltpu.VMEM((RB, B), jnp.int32),
            pltpu.VMEM((STRIPE,), jnp.float32),
        ]
        + [pltpu.VMEM((B,), jnp.float32) for _ in range(ncol)]
        + [pltpu.SemaphoreType.DMA]
        + [pltpu.VMEM_SHARED((NP,), jnp.float32) for _ in range(2 * ncol)],
    )
    def agg(src_hbm, dst_hbm, *rest):
        tabs_hbm = rest[:ncol]
        zeros_hbm = rest[ncol]
        outs_hbm = rest[ncol + 1 : 2 * ncol + 1]
        sstage, dstage, bounce = rest[2 * ncol + 1 : 2 * ncol + 4]
        vals = rest[2 * ncol + 4 : 3 * ncol + 4]
        sem = rest[3 * ncol + 4]
        tabs_sh = rest[3 * ncol + 5 : 4 * ncol + 5]
        aggs_sh = rest[4 * ncol + 5 : 5 * ncol + 5]

        c = lax.axis_index("c")
        s = lax.axis_index("s")
        stripe = pl.ds(s * STRIPE, STRIPE)
        pltpu.sync_copy(zeros_hbm.at[stripe], bounce)
        for k in range(ncol):
            pltpu.sync_copy(bounce, aggs_sh[k].at[stripe])
        for k in range(ncol):
            pltpu.sync_copy(tabs_hbm[k].at[stripe], bounce)
            pltpu.sync_copy(bounce, tabs_sh[k].at[stripe])
        plsc.subcore_barrier()

        wid = c * NS + s
        trips = NBF + jnp.where(wid < NBR, 1, 0)

        def blk(b, carry):
            base = (wid + b * NW) * RB
            pltpu.sync_copy(src_hbm.at[pl.ds(base, RB)], sstage)
            pltpu.sync_copy(dst_hbm.at[pl.ds(base, RB)], dstage)

            def row(r, carry2):
                descs = [
                    pltpu.async_copy(tabs_sh[k].at[sstage.at[r]], vals[k], sem)
                    for k in range(ncol)
                ]
                for d in descs:
                    d.wait()
                for k in range(ncol):
                    pltpu.sync_copy(
                        vals[k], aggs_sh[k].at[dstage.at[r]], add=True
                    )
                return carry2

            return lax.fori_loop(0, RB, row, carry)

        lax.fori_loop(0, trips, blk, 0)
        plsc.subcore_barrier()
        out_off = c * NP + s * STRIPE
        for k in range(ncol):
            pltpu.sync_copy(aggs_sh[k].at[stripe], bounce)
            pltpu.sync_copy(bounce, outs_hbm[k].at[pl.ds(out_off, STRIPE)])

    return agg


@functools.partial(
    pl.kernel,
    out_type=jax.ShapeDtypeStruct((NC * NP,), jnp.float32),
    mesh=_mesh,
    scratch_types=[
        pltpu.VMEM((RB, B), jnp.int32),
        pltpu.VMEM((STRIPE,), jnp.float32),
        pltpu.VMEM((B,), jnp.float32),
        pltpu.VMEM_SHARED((NP,), jnp.float32),
    ],
)
def _sc_degree(dst_hbm, zeros_hbm, out_hbm, dstage, bounce, ones, deg_sh):
    c = lax.axis_index("c")
    s = lax.axis_index("s")
    stripe = pl.ds(s * STRIPE, STRIPE)
    pltpu.sync_copy(zeros_hbm.at[stripe], bounce)
    pltpu.sync_copy(bounce, deg_sh.at[stripe])
    for i in range(B // 16):
        ones[pl.ds(i * 16, 16)] = jnp.ones((16,), jnp.float32)
    plsc.subcore_barrier()

    wid = c * NS + s
    trips = NBF + jnp.where(wid < NBR, 1, 0)

    def blk(b, carry):
        base = (wid + b * NW) * RB
        pltpu.sync_copy(dst_hbm.at[pl.ds(base, RB)], dstage)

        def row(r, carry2):
            pltpu.sync_copy(ones, deg_sh.at[dstage.at[r]], add=True)
            return carry2

        return lax.fori_loop(0, RB, row, carry)

    lax.fori_loop(0, trips, blk, 0)
    plsc.subcore_barrier()
    pltpu.sync_copy(deg_sh.at[stripe], bounce)
    pltpu.sync_copy(bounce, out_hbm.at[pl.ds(c * NP + s * STRIPE, STRIPE)])


_sc_agg3 = _make_sc_agg(3)
_sc_agg1 = _make_sc_agg(1)


def _tc_prep_body(degp, x0, x1, x2, dinv_o, y0_o, y1_o, y2_o):
    deg = degp[pl.ds(0, NP)] + degp[pl.ds(NP, NP)] + 1.0
    dinv = lax.rsqrt(deg)
    dinv_o[...] = dinv
    y0_o[...] = x0[...] * dinv
    y1_o[...] = x1[...] * dinv
    y2_o[...] = x2[...] * dinv


_tc_prep = pl.pallas_call(
    _tc_prep_body,
    out_shape=[jax.ShapeDtypeStruct((NP,), jnp.float32)] * 4,
)


def _tc_mid_body(a0p, a1p, a2p, y0, y1, y2, dinv_i, W1, b1, W2, z_o):
    dinv = dinv_i[...]
    t0 = dinv * (a0p[pl.ds(0, NP)] + a0p[pl.ds(NP, NP)] + y0[...])
    t1 = dinv * (a1p[pl.ds(0, NP)] + a1p[pl.ds(NP, NP)] + y1[...])
    t2 = dinv * (a2p[pl.ds(0, NP)] + a2p[pl.ds(NP, NP)] + y2[...])
    acc = jnp.zeros((NP,), jnp.float32)
    for j in range(16):
        hj = t0 * W1[0, j] + t1 * W1[1, j] + t2 * W1[2, j] + b1[j]
        acc = acc + jnp.maximum(hj, 0.0) * W2[j, 0]
    z_o[...] = dinv * acc


_tc_mid = pl.pallas_call(
    _tc_mid_body,
    in_specs=[pl.BlockSpec()] * 7
    + [pl.BlockSpec(memory_space=pltpu.SMEM)] * 3,
    out_shape=jax.ShapeDtypeStruct((NP,), jnp.float32),
)


def _tc_final_body(zp, z, dinv, b2, out_o):
    out_o[...] = dinv[...] * (zp[pl.ds(0, NP)] + zp[pl.ds(NP, NP)] + z[...]) + b2[0]


_tc_final = pl.pallas_call(
    _tc_final_body,
    in_specs=[pl.BlockSpec()] * 3 + [pl.BlockSpec(memory_space=pltpu.SMEM)],
    out_shape=jax.ShapeDtypeStruct((NP,), jnp.float32),
)


def kernel(x, edge_index, W1, b1, W2, b2):
    ei = edge_index.astype(jnp.int32)
    src2d = ei[0].reshape(ROWS, B)
    dst2d = ei[1].reshape(ROWS, B)
    xp = jnp.pad(x.astype(jnp.float32), ((0, NP - N), (0, 0)))
    x0, x1, x2 = xp[:, 0], xp[:, 1], xp[:, 2]
    zeros_np = jnp.zeros((NP,), jnp.float32)

    degp = _sc_degree(dst2d, zeros_np)
    dinv, y0, y1, y2 = _tc_prep(degp, x0, x1, x2)
    a0p, a1p, a2p = _sc_agg3(src2d, dst2d, y0, y1, y2, zeros_np)
    z = _tc_mid(a0p, a1p, a2p, y0, y1, y2, dinv, W1, b1, W2)
    (zp,) = _sc_agg1(src2d, dst2d, z, zeros_np)
    outp = _tc_final(zp, z, dinv, b2)
    return outp[:N]


# speedup vs baseline: 187.7282x; 2.0147x over previous
import functools

import jax
import jax.numpy as jnp
from jax import lax
from jax.experimental import pallas as pl
from jax.experimental.pallas import tpu as pltpu
from jax.experimental.pallas import tpu_sc as plsc

N = 100000
E = 6400000
NP = 100096
B = 128
ROWS = E // B
NC = 2
NS = 16
NW = NC * NS
RB = 16
NBT = ROWS // RB
NBF = NBT // NW
NBR = NBT - NBF * NW
STRIPE = NP // NS

_mesh = plsc.VectorSubcoreMesh(core_axis_name="c", subcore_axis_name="s")


def _make_sc_agg(ncol):
    @functools.partial(
        pl.kernel,
        out_type=[jax.ShapeDtypeStruct((NC * NP,), jnp.float32)] * ncol,
        mesh=_mesh,
        scratch_types=[
            pltpu.VMEM((RB, B), jnp.int32),
            pltpu.VMEM((RB, B), jnp.int32),
            pltpu.VMEM((STRIPE,), jnp.float32),
        ]
        + [pltpu.VMEM((RB, B), jnp.float32) for _ in range(ncol)]
        + [pltpu.SemaphoreType.DMA, pltpu.SemaphoreType.DMA]
        + [pltpu.VMEM_SHARED((NP,), jnp.float32) for _ in range(2 * ncol)],
    )
    def agg(src_hbm, dst_hbm, *rest):
        tabs_hbm = rest[:ncol]
        zeros_hbm = rest[ncol]
        outs_hbm = rest[ncol + 1 : 2 * ncol + 1]
        sstage, dstage, bounce = rest[2 * ncol + 1 : 2 * ncol + 4]
        vals = rest[2 * ncol + 4 : 3 * ncol + 4]
        sem_g, sem_s = rest[3 * ncol + 4 : 3 * ncol + 6]
        tabs_sh = rest[3 * ncol + 6 : 4 * ncol + 6]
        aggs_sh = rest[4 * ncol + 6 : 5 * ncol + 6]

        c = lax.axis_index("c")
        s = lax.axis_index("s")
        stripe = pl.ds(s * STRIPE, STRIPE)
        pltpu.sync_copy(zeros_hbm.at[stripe], bounce)
        for k in range(ncol):
            pltpu.sync_copy(bounce, aggs_sh[k].at[stripe])
        for k in range(ncol):
            pltpu.sync_copy(tabs_hbm[k].at[stripe], bounce)
            pltpu.sync_copy(bounce, tabs_sh[k].at[stripe])
        plsc.subcore_barrier()

        wid = c * NS + s
        trips = NBF + jnp.where(wid < NBR, 1, 0)
        b0 = wid * NBF + jnp.minimum(wid, NBR)

        def blk(b, carry):
            base = (b0 + b) * RB
            pltpu.sync_copy(src_hbm.at[pl.ds(base, RB)], sstage)
            pltpu.sync_copy(dst_hbm.at[pl.ds(base, RB)], dstage)
            gd = [
                pltpu.async_copy(
                    tabs_sh[k].at[sstage.at[r]], vals[k].at[r], sem_g
                )
                for r in range(RB)
                for k in range(ncol)
            ]
            for d in gd:
                d.wait()
            sd = [
                pltpu.async_copy(
                    vals[k].at[r], aggs_sh[k].at[dstage.at[r]], sem_s, add=True
                )
                for r in range(RB)
                for k in range(ncol)
            ]
            for d in sd:
                d.wait()
            return carry

        lax.fori_loop(0, trips, blk, 0)
        plsc.subcore_barrier()
        out_off = c * NP + s * STRIPE
        for k in range(ncol):
            pltpu.sync_copy(aggs_sh[k].at[stripe], bounce)
            pltpu.sync_copy(bounce, outs_hbm[k].at[pl.ds(out_off, STRIPE)])

    return agg


@functools.partial(
    pl.kernel,
    out_type=jax.ShapeDtypeStruct((NC * NP,), jnp.float32),
    mesh=_mesh,
    scratch_types=[
        pltpu.VMEM((RB, B), jnp.int32),
        pltpu.VMEM((STRIPE,), jnp.float32),
        pltpu.VMEM((B,), jnp.float32),
        pltpu.SemaphoreType.DMA,
        pltpu.VMEM_SHARED((NP,), jnp.float32),
    ],
)
def _sc_degree(dst_hbm, zeros_hbm, out_hbm, dstage, bounce, ones, sem_s, deg_sh):
    c = lax.axis_index("c")
    s = lax.axis_index("s")
    stripe = pl.ds(s * STRIPE, STRIPE)
    pltpu.sync_copy(zeros_hbm.at[stripe], bounce)
    pltpu.sync_copy(bounce, deg_sh.at[stripe])
    for i in range(B // 16):
        ones[pl.ds(i * 16, 16)] = jnp.ones((16,), jnp.float32)
    plsc.subcore_barrier()

    wid = c * NS + s
    trips = NBF + jnp.where(wid < NBR, 1, 0)
    b0 = wid * NBF + jnp.minimum(wid, NBR)

    def blk(b, carry):
        base = (b0 + b) * RB
        pltpu.sync_copy(dst_hbm.at[pl.ds(base, RB)], dstage)
        sd = [
            pltpu.async_copy(ones, deg_sh.at[dstage.at[r]], sem_s, add=True)
            for r in range(RB)
        ]
        for d in sd:
            d.wait()
        return carry

    lax.fori_loop(0, trips, blk, 0)
    plsc.subcore_barrier()
    pltpu.sync_copy(deg_sh.at[stripe], bounce)
    pltpu.sync_copy(bounce, out_hbm.at[pl.ds(c * NP + s * STRIPE, STRIPE)])


_sc_agg3 = _make_sc_agg(3)
_sc_agg1 = _make_sc_agg(1)


def _tc_prep_body(degp, x0, x1, x2, dinv_o, y0_o, y1_o, y2_o):
    deg = degp[pl.ds(0, NP)] + degp[pl.ds(NP, NP)] + 1.0
    dinv = lax.rsqrt(deg)
    dinv_o[...] = dinv
    y0_o[...] = x0[...] * dinv
    y1_o[...] = x1[...] * dinv
    y2_o[...] = x2[...] * dinv


_tc_prep = pl.pallas_call(
    _tc_prep_body,
    out_shape=[jax.ShapeDtypeStruct((NP,), jnp.float32)] * 4,
)


def _tc_mid_body(a0p, a1p, a2p, y0, y1, y2, dinv_i, W1, b1, W2, z_o):
    dinv = dinv_i[...]
    t0 = dinv * (a0p[pl.ds(0, NP)] + a0p[pl.ds(NP, NP)] + y0[...])
    t1 = dinv * (a1p[pl.ds(0, NP)] + a1p[pl.ds(NP, NP)] + y1[...])
    t2 = dinv * (a2p[pl.ds(0, NP)] + a2p[pl.ds(NP, NP)] + y2[...])
    acc = jnp.zeros((NP,), jnp.float32)
    for j in range(16):
        hj = t0 * W1[0, j] + t1 * W1[1, j] + t2 * W1[2, j] + b1[j]
        acc = acc + jnp.maximum(hj, 0.0) * W2[j, 0]
    z_o[...] = dinv * acc


_tc_mid = pl.pallas_call(
    _tc_mid_body,
    in_specs=[pl.BlockSpec()] * 7
    + [pl.BlockSpec(memory_space=pltpu.SMEM)] * 3,
    out_shape=jax.ShapeDtypeStruct((NP,), jnp.float32),
)


def _tc_final_body(zp, z, dinv, b2, out_o):
    out_o[...] = dinv[...] * (zp[pl.ds(0, NP)] + zp[pl.ds(NP, NP)] + z[...]) + b2[0]


_tc_final = pl.pallas_call(
    _tc_final_body,
    in_specs=[pl.BlockSpec()] * 3 + [pl.BlockSpec(memory_space=pltpu.SMEM)],
    out_shape=jax.ShapeDtypeStruct((NP,), jnp.float32),
)


def kernel(x, edge_index, W1, b1, W2, b2):
    ei = edge_index.astype(jnp.int32)
    src2d = ei[0].reshape(ROWS, B)
    dst2d = ei[1].reshape(ROWS, B)
    xp = jnp.pad(x.astype(jnp.float32), ((0, NP - N), (0, 0)))
    x0, x1, x2 = xp[:, 0], xp[:, 1], xp[:, 2]
    zeros_np = jnp.zeros((NP,), jnp.float32)

    degp = _sc_degree(dst2d, zeros_np)
    dinv, y0, y1, y2 = _tc_prep(degp, x0, x1, x2)
    a0p, a1p, a2p = _sc_agg3(src2d, dst2d, y0, y1, y2, zeros_np)
    z = _tc_mid(a0p, a1p, a2p, y0, y1, y2, dinv, W1, b1, W2)
    (zp,) = _sc_agg1(src2d, dst2d, z, zeros_np)
    outp = _tc_final(zp, z, dinv, b2)
    return outp[:N]


# speedup vs baseline: 224.2219x; 1.1944x over previous
import functools

import jax
import jax.numpy as jnp
from jax import lax
from jax.experimental import pallas as pl
from jax.experimental.pallas import tpu as pltpu
from jax.experimental.pallas import tpu_sc as plsc

N = 100000
E = 6400000
NP = 100096
B = 128
ROWS = E // B
NC = 2
NS = 16
NW = NC * NS
RB = 16
NBT = ROWS // RB
NBF = NBT // NW
NBR = NBT - NBF * NW
STRIPE = NP // NS

_mesh = plsc.VectorSubcoreMesh(core_axis_name="c", subcore_axis_name="s")


def _make_sc_agg(ncol):
    @functools.partial(
        pl.kernel,
        out_type=[jax.ShapeDtypeStruct((NC * NP,), jnp.float32)] * ncol,
        mesh=_mesh,
        scratch_types=[
            pltpu.VMEM((RB, B), jnp.int32),
            pltpu.VMEM((RB, B), jnp.int32),
            pltpu.VMEM((STRIPE,), jnp.float32),
        ]
        + [pltpu.VMEM((RB, B), jnp.float32) for _ in range(ncol)]
        + [pltpu.SemaphoreType.DMA((RB,)), pltpu.SemaphoreType.DMA,
           pltpu.SemaphoreType.DMA, pltpu.SemaphoreType.DMA]
        + [pltpu.VMEM_SHARED((NP,), jnp.float32) for _ in range(2 * ncol)],
    )
    def agg(src_hbm, dst_hbm, *rest):
        tabs_hbm = rest[:ncol]
        zeros_hbm = rest[ncol]
        outs_hbm = rest[ncol + 1 : 2 * ncol + 1]
        sstage, dstage, bounce = rest[2 * ncol + 1 : 2 * ncol + 4]
        vals = rest[2 * ncol + 4 : 3 * ncol + 4]
        sem_g, sem_s, sem_ts, sem_td = rest[3 * ncol + 4 : 3 * ncol + 8]
        tabs_sh = rest[3 * ncol + 8 : 4 * ncol + 8]
        aggs_sh = rest[4 * ncol + 8 : 5 * ncol + 8]

        c = lax.axis_index("c")
        s = lax.axis_index("s")
        stripe = pl.ds(s * STRIPE, STRIPE)
        pltpu.sync_copy(zeros_hbm.at[stripe], bounce)
        for k in range(ncol):
            pltpu.sync_copy(bounce, aggs_sh[k].at[stripe])
        for k in range(ncol):
            pltpu.sync_copy(tabs_hbm[k].at[stripe], bounce)
            pltpu.sync_copy(bounce, tabs_sh[k].at[stripe])
        plsc.subcore_barrier()

        wid = c * NS + s
        trips = NBF + jnp.where(wid < NBR, 1, 0)
        b0 = wid * NBF + jnp.minimum(wid, NBR)

        pltpu.async_copy(src_hbm.at[pl.ds(b0 * RB, RB)], sstage, sem_ts)
        pltpu.async_copy(dst_hbm.at[pl.ds(b0 * RB, RB)], dstage, sem_td)

        def blk(b, carry):
            pltpu.make_async_copy(
                src_hbm.at[pl.ds(0, RB)], sstage, sem_ts
            ).wait()
            pltpu.make_async_copy(
                dst_hbm.at[pl.ds(0, RB)], dstage, sem_td
            ).wait()
            gd = [
                [
                    pltpu.async_copy(
                        tabs_sh[k].at[sstage.at[r]], vals[k].at[r],
                        sem_g.at[r],
                    )
                    for k in range(ncol)
                ]
                for r in range(RB)
            ]
            sd = []
            for r in range(RB):
                for d in gd[r]:
                    d.wait()
                for k in range(ncol):
                    sd.append(
                        pltpu.async_copy(
                            vals[k].at[r], aggs_sh[k].at[dstage.at[r]],
                            sem_s, add=True,
                        )
                    )
            @pl.when(b + 1 < trips)
            def _():
                pltpu.async_copy(
                    src_hbm.at[pl.ds((b0 + b + 1) * RB, RB)], sstage, sem_ts
                )

            for d in sd:
                d.wait()

            @pl.when(b + 1 < trips)
            def _():
                pltpu.async_copy(
                    dst_hbm.at[pl.ds((b0 + b + 1) * RB, RB)], dstage, sem_td
                )

            return carry

        lax.fori_loop(0, trips, blk, 0)
        plsc.subcore_barrier()
        out_off = c * NP + s * STRIPE
        for k in range(ncol):
            pltpu.sync_copy(aggs_sh[k].at[stripe], bounce)
            pltpu.sync_copy(bounce, outs_hbm[k].at[pl.ds(out_off, STRIPE)])

    return agg


@functools.partial(
    pl.kernel,
    out_type=jax.ShapeDtypeStruct((NC * NP,), jnp.float32),
    mesh=_mesh,
    scratch_types=[
        pltpu.VMEM((RB, B), jnp.int32),
        pltpu.VMEM((STRIPE,), jnp.float32),
        pltpu.VMEM((B,), jnp.float32),
        pltpu.SemaphoreType.DMA,
        pltpu.VMEM_SHARED((NP,), jnp.float32),
    ],
)
def _sc_degree(dst_hbm, zeros_hbm, out_hbm, dstage, bounce, ones, sem_s, deg_sh):
    c = lax.axis_index("c")
    s = lax.axis_index("s")
    stripe = pl.ds(s * STRIPE, STRIPE)
    pltpu.sync_copy(zeros_hbm.at[stripe], bounce)
    pltpu.sync_copy(bounce, deg_sh.at[stripe])
    for i in range(B // 16):
        ones[pl.ds(i * 16, 16)] = jnp.ones((16,), jnp.float32)
    plsc.subcore_barrier()

    wid = c * NS + s
    trips = NBF + jnp.where(wid < NBR, 1, 0)
    b0 = wid * NBF + jnp.minimum(wid, NBR)

    def blk(b, carry):
        base = (b0 + b) * RB
        pltpu.sync_copy(dst_hbm.at[pl.ds(base, RB)], dstage)
        sd = [
            pltpu.async_copy(ones, deg_sh.at[dstage.at[r]], sem_s, add=True)
            for r in range(RB)
        ]
        for d in sd:
            d.wait()
        return carry

    lax.fori_loop(0, trips, blk, 0)
    plsc.subcore_barrier()
    pltpu.sync_copy(deg_sh.at[stripe], bounce)
    pltpu.sync_copy(bounce, out_hbm.at[pl.ds(c * NP + s * STRIPE, STRIPE)])


_sc_agg3 = _make_sc_agg(3)
_sc_agg1 = _make_sc_agg(1)


def _tc_prep_body(degp, x0, x1, x2, dinv_o, y0_o, y1_o, y2_o):
    deg = degp[pl.ds(0, NP)] + degp[pl.ds(NP, NP)] + 1.0
    dinv = lax.rsqrt(deg)
    dinv_o[...] = dinv
    y0_o[...] = x0[...] * dinv
    y1_o[...] = x1[...] * dinv
    y2_o[...] = x2[...] * dinv


_tc_prep = pl.pallas_call(
    _tc_prep_body,
    out_shape=[jax.ShapeDtypeStruct((NP,), jnp.float32)] * 4,
)


def _tc_mid_body(a0p, a1p, a2p, y0, y1, y2, dinv_i, W1, b1, W2, z_o):
    dinv = dinv_i[...]
    t0 = dinv * (a0p[pl.ds(0, NP)] + a0p[pl.ds(NP, NP)] + y0[...])
    t1 = dinv * (a1p[pl.ds(0, NP)] + a1p[pl.ds(NP, NP)] + y1[...])
    t2 = dinv * (a2p[pl.ds(0, NP)] + a2p[pl.ds(NP, NP)] + y2[...])
    acc = jnp.zeros((NP,), jnp.float32)
    for j in range(16):
        hj = t0 * W1[0, j] + t1 * W1[1, j] + t2 * W1[2, j] + b1[j]
        acc = acc + jnp.maximum(hj, 0.0) * W2[j, 0]
    z_o[...] = dinv * acc


_tc_mid = pl.pallas_call(
    _tc_mid_body,
    in_specs=[pl.BlockSpec()] * 7
    + [pl.BlockSpec(memory_space=pltpu.SMEM)] * 3,
    out_shape=jax.ShapeDtypeStruct((NP,), jnp.float32),
)


def _tc_final_body(zp, z, dinv, b2, out_o):
    out_o[...] = dinv[...] * (zp[pl.ds(0, NP)] + zp[pl.ds(NP, NP)] + z[...]) + b2[0]


_tc_final = pl.pallas_call(
    _tc_final_body,
    in_specs=[pl.BlockSpec()] * 3 + [pl.BlockSpec(memory_space=pltpu.SMEM)],
    out_shape=jax.ShapeDtypeStruct((NP,), jnp.float32),
)


def kernel(x, edge_index, W1, b1, W2, b2):
    ei = edge_index.astype(jnp.int32)
    src2d = ei[0].reshape(ROWS, B)
    dst2d = ei[1].reshape(ROWS, B)
    xp = jnp.pad(x.astype(jnp.float32), ((0, NP - N), (0, 0)))
    x0, x1, x2 = xp[:, 0], xp[:, 1], xp[:, 2]
    zeros_np = jnp.zeros((NP,), jnp.float32)

    degp = _sc_degree(dst2d, zeros_np)
    dinv, y0, y1, y2 = _tc_prep(degp, x0, x1, x2)
    a0p, a1p, a2p = _sc_agg3(src2d, dst2d, y0, y1, y2, zeros_np)
    z = _tc_mid(a0p, a1p, a2p, y0, y1, y2, dinv, W1, b1, W2)
    (zp,) = _sc_agg1(src2d, dst2d, z, zeros_np)
    outp = _tc_final(zp, z, dinv, b2)
    return outp[:N]


# speedup vs baseline: 239.5933x; 1.0686x over previous
import functools

import jax
import jax.numpy as jnp
from jax import lax
from jax.experimental import pallas as pl
from jax.experimental.pallas import tpu as pltpu
from jax.experimental.pallas import tpu_sc as plsc

N = 100000
E = 6400000
NP = 100096
B = 128
ROWS = E // B
NC = 2
NS = 16
NW = NC * NS
RB = 16
NBT = ROWS // RB
NBF = NBT // NW
NBR = NBT - NBF * NW
STRIPE = NP // NS
STRIPE2 = STRIPE // 2

_mesh = plsc.VectorSubcoreMesh(core_axis_name="c", subcore_axis_name="s")


def _make_sc_vg(ncol):
    @functools.partial(
        pl.kernel,
        out_type=[jax.ShapeDtypeStruct((NC * NP,), jnp.float32)] * ncol,
        mesh=_mesh,
        compiler_params=pltpu.CompilerParams(needs_layout_passes=False),
        scratch_types=[
            pltpu.VMEM((RB, B), jnp.int32),
            pltpu.VMEM((RB, B), jnp.int32),
            pltpu.VMEM((STRIPE2,), jnp.float32),
            pltpu.VMEM((RB, B), jnp.float32),
            pltpu.VMEM((NP,), jnp.float32),
            pltpu.SemaphoreType.DMA,
            pltpu.SemaphoreType.DMA,
            pltpu.SemaphoreType.DMA,
        ]
        + [pltpu.VMEM_SHARED((NP,), jnp.float32) for _ in range(ncol)],
    )
    def vg(src_hbm, dst_hbm, *rest):
        tabs_hbm = rest[:ncol]
        zeros_hbm = rest[ncol]
        outs_hbm = rest[ncol + 1 : 2 * ncol + 1]
        (sstage, dstage, bounce, vals, ytab,
         sem_s, sem_ts, sem_td) = rest[2 * ncol + 1 : 2 * ncol + 9]
        aggs_sh = rest[2 * ncol + 9 :]

        c = lax.axis_index("c")
        s = lax.axis_index("s")
        pltpu.sync_copy(zeros_hbm.at[pl.ds(s * STRIPE, STRIPE2)], bounce)
        for k in range(ncol):
            for h in range(2):
                pltpu.sync_copy(
                    bounce,
                    aggs_sh[k].at[pl.ds(s * STRIPE + h * STRIPE2, STRIPE2)],
                )
        plsc.subcore_barrier()

        wid = c * NS + s
        trips = NBF + jnp.where(wid < NBR, 1, 0)
        b0 = wid * NBF + jnp.minimum(wid, NBR)

        for k in range(ncol):
            pltpu.sync_copy(tabs_hbm[k], ytab)
            pltpu.async_copy(src_hbm.at[pl.ds(b0 * RB, RB)], sstage, sem_ts)
            pltpu.async_copy(dst_hbm.at[pl.ds(b0 * RB, RB)], dstage, sem_td)

            def blk(b, carry):
                pltpu.make_async_copy(
                    src_hbm.at[pl.ds(0, RB)], sstage, sem_ts
                ).wait()
                pltpu.make_async_copy(
                    dst_hbm.at[pl.ds(0, RB)], dstage, sem_td
                ).wait()
                sd = []
                for r in range(RB):
                    for g in range(B // 16):
                        sl = pl.ds(g * 16, 16)
                        idxv = sstage[r, sl]
                        vals[r, sl] = plsc.load_gather(ytab, [idxv])
                    sd.append(
                        pltpu.async_copy(
                            vals.at[r], aggs_sh[k].at[dstage.at[r]],
                            sem_s, add=True,
                        )
                    )

                @pl.when(b + 1 < trips)
                def _():
                    pltpu.async_copy(
                        src_hbm.at[pl.ds((b0 + b + 1) * RB, RB)],
                        sstage, sem_ts,
                    )

                for d in sd:
                    d.wait()

                @pl.when(b + 1 < trips)
                def _():
                    pltpu.async_copy(
                        dst_hbm.at[pl.ds((b0 + b + 1) * RB, RB)],
                        dstage, sem_td,
                    )

                return carry

            lax.fori_loop(0, trips, blk, 0)

        plsc.subcore_barrier()
        for k in range(ncol):
            for h in range(2):
                off = s * STRIPE + h * STRIPE2
                pltpu.sync_copy(aggs_sh[k].at[pl.ds(off, STRIPE2)], bounce)
                pltpu.sync_copy(
                    bounce, outs_hbm[k].at[pl.ds(c * NP + off, STRIPE2)]
                )

    return vg


@functools.partial(
    pl.kernel,
    out_type=jax.ShapeDtypeStruct((NC * NP,), jnp.float32),
    mesh=_mesh,
    scratch_types=[
        pltpu.VMEM((RB, B), jnp.int32),
        pltpu.VMEM((STRIPE,), jnp.float32),
        pltpu.VMEM((B,), jnp.float32),
        pltpu.SemaphoreType.DMA,
        pltpu.SemaphoreType.DMA,
        pltpu.VMEM_SHARED((NP,), jnp.float32),
    ],
)
def _sc_degree(dst_hbm, zeros_hbm, out_hbm, dstage, bounce, ones,
               sem_s, sem_td, deg_sh):
    c = lax.axis_index("c")
    s = lax.axis_index("s")
    stripe = pl.ds(s * STRIPE, STRIPE)
    pltpu.sync_copy(zeros_hbm.at[stripe], bounce)
    pltpu.sync_copy(bounce, deg_sh.at[stripe])
    for i in range(B // 16):
        ones[pl.ds(i * 16, 16)] = jnp.ones((16,), jnp.float32)
    plsc.subcore_barrier()

    wid = c * NS + s
    trips = NBF + jnp.where(wid < NBR, 1, 0)
    b0 = wid * NBF + jnp.minimum(wid, NBR)
    pltpu.async_copy(dst_hbm.at[pl.ds(b0 * RB, RB)], dstage, sem_td)

    def blk(b, carry):
        pltpu.make_async_copy(dst_hbm.at[pl.ds(0, RB)], dstage, sem_td).wait()
        sd = [
            pltpu.async_copy(ones, deg_sh.at[dstage.at[r]], sem_s, add=True)
            for r in range(RB)
        ]
        for d in sd:
            d.wait()

        @pl.when(b + 1 < trips)
        def _():
            pltpu.async_copy(
                dst_hbm.at[pl.ds((b0 + b + 1) * RB, RB)], dstage, sem_td
            )

        return carry

    lax.fori_loop(0, trips, blk, 0)
    plsc.subcore_barrier()
    pltpu.sync_copy(deg_sh.at[stripe], bounce)
    pltpu.sync_copy(bounce, out_hbm.at[pl.ds(c * NP + s * STRIPE, STRIPE)])


_sc_agg3 = _make_sc_vg(3)
_sc_agg1 = _make_sc_vg(1)


def _tc_prep_body(degp, x0, x1, x2, dinv_o, y0_o, y1_o, y2_o):
    deg = degp[pl.ds(0, NP)] + degp[pl.ds(NP, NP)] + 1.0
    dinv = lax.rsqrt(deg)
    dinv_o[...] = dinv
    y0_o[...] = x0[...] * dinv
    y1_o[...] = x1[...] * dinv
    y2_o[...] = x2[...] * dinv


_tc_prep = pl.pallas_call(
    _tc_prep_body,
    out_shape=[jax.ShapeDtypeStruct((NP,), jnp.float32)] * 4,
)


def _tc_mid_body(a0p, a1p, a2p, y0, y1, y2, dinv_i, W1, b1, W2, z_o):
    dinv = dinv_i[...]
    t0 = dinv * (a0p[pl.ds(0, NP)] + a0p[pl.ds(NP, NP)] + y0[...])
    t1 = dinv * (a1p[pl.ds(0, NP)] + a1p[pl.ds(NP, NP)] + y1[...])
    t2 = dinv * (a2p[pl.ds(0, NP)] + a2p[pl.ds(NP, NP)] + y2[...])
    acc = jnp.zeros((NP,), jnp.float32)
    for j in range(16):
        hj = t0 * W1[0, j] + t1 * W1[1, j] + t2 * W1[2, j] + b1[j]
        acc = acc + jnp.maximum(hj, 0.0) * W2[j, 0]
    z_o[...] = dinv * acc


_tc_mid = pl.pallas_call(
    _tc_mid_body,
    in_specs=[pl.BlockSpec()] * 7
    + [pl.BlockSpec(memory_space=pltpu.SMEM)] * 3,
    out_shape=jax.ShapeDtypeStruct((NP,), jnp.float32),
)


def _tc_final_body(zp, z, dinv, b2, out_o):
    out_o[...] = dinv[...] * (zp[pl.ds(0, NP)] + zp[pl.ds(NP, NP)] + z[...]) + b2[0]


_tc_final = pl.pallas_call(
    _tc_final_body,
    in_specs=[pl.BlockSpec()] * 3 + [pl.BlockSpec(memory_space=pltpu.SMEM)],
    out_shape=jax.ShapeDtypeStruct((NP,), jnp.float32),
)


def kernel(x, edge_index, W1, b1, W2, b2):
    ei = edge_index.astype(jnp.int32)
    src2d = ei[0].reshape(ROWS, B)
    dst2d = ei[1].reshape(ROWS, B)
    xp = jnp.pad(x.astype(jnp.float32), ((0, NP - N), (0, 0)))
    x0, x1, x2 = xp[:, 0], xp[:, 1], xp[:, 2]
    zeros_np = jnp.zeros((NP,), jnp.float32)

    degp = _sc_degree(dst2d, zeros_np)
    dinv, y0, y1, y2 = _tc_prep(degp, x0, x1, x2)
    a0p, a1p, a2p = _sc_agg3(src2d, dst2d, y0, y1, y2, zeros_np)
    z = _tc_mid(a0p, a1p, a2p, y0, y1, y2, dinv, W1, b1, W2)
    (zp,) = _sc_agg1(src2d, dst2d, z, zeros_np)
    outp = _tc_final(zp, z, dinv, b2)
    return outp[:N]
